# baseline (speedup 1.0000x reference)
import jax
import jax.numpy as jnp
from jax.experimental import pallas as pl
from jax.experimental.pallas import tpu as pltpu

HIDDEN = 2048
NUM_EXPERTS = 16
TOKENS = 16384
BLOCK = 2048
M_SPLIT = 1536
LANES = 128
KCHUNKS = HIDDEN // LANES


def _top2(logits):
    lane = jax.lax.broadcasted_iota(jnp.int32, logits.shape, 1)
    m1 = jnp.max(logits, axis=1, keepdims=True)
    i1 = jnp.min(jnp.where(logits == m1, lane, NUM_EXPERTS), axis=1,
                 keepdims=True)
    masked = jnp.where(lane == i1, -jnp.inf, logits)
    m2 = jnp.max(masked, axis=1, keepdims=True)
    i2 = jnp.min(jnp.where(masked == m2, lane, NUM_EXPERTS), axis=1,
                 keepdims=True)
    r = jnp.exp(m2 - m1)
    w1 = 1.0 / (1.0 + r)
    return (jnp.concatenate([w1, 1.0 - w1], axis=1),
            jnp.concatenate([i1, i2], axis=1))


def _router_kernel(x_ref, wt_ref, w_ref, w_out_ref, i_out_ref):
    lm = jnp.dot(x_ref[0:M_SPLIT, :], wt_ref[...],
                 preferred_element_type=jnp.float32)
    xv = x_ref[M_SPLIT:BLOCK, :].astype(jnp.bfloat16).astype(jnp.float32)
    wv = w_ref[...].astype(jnp.bfloat16).astype(jnp.float32)
    cols = []
    for e in range(NUM_EXPERTS):
        acc = xv[:, 0:LANES] * wv[e:e + 1, 0:LANES]
        for c in range(1, KCHUNKS):
            acc = acc + xv[:, c * LANES:(c + 1) * LANES] * \
                wv[e:e + 1, c * LANES:(c + 1) * LANES]
        cols.append(jnp.sum(acc, axis=1, keepdims=True))
    lv = jnp.concatenate(cols, axis=1)
    logits = jnp.concatenate([lm, lv], axis=0)
    w_out, i_out = _top2(logits)
    w_out_ref[...] = w_out
    i_out_ref[...] = i_out


@jax.jit
def kernel(x, W):
    grid = (TOKENS // BLOCK,)
    w_out, i_out = pl.pallas_call(
        _router_kernel,
        grid=grid,
        in_specs=[
            pl.BlockSpec((BLOCK, HIDDEN), lambda i: (i, 0)),
            pl.BlockSpec((HIDDEN, NUM_EXPERTS), lambda i: (0, 0)),
            pl.BlockSpec((NUM_EXPERTS, HIDDEN), lambda i: (0, 0)),
        ],
        out_specs=[
            pl.BlockSpec((BLOCK, 2), lambda i: (i, 0)),
            pl.BlockSpec((BLOCK, 2), lambda i: (i, 0)),
        ],
        out_shape=[
            jax.ShapeDtypeStruct((TOKENS, 2), jnp.float32),
            jax.ShapeDtypeStruct((TOKENS, 2), jnp.int32),
        ],
        compiler_params=pltpu.CompilerParams(
            dimension_semantics=("arbitrary",),
        ),
    )(x, W.T, W)
    return (w_out, i_out)


# speedup vs baseline: 1.1579x; 1.1579x over previous
import functools

import jax
import jax.numpy as jnp
from jax.experimental import pallas as pl
from jax.experimental.pallas import tpu as pltpu

HIDDEN = 2048
NUM_EXPERTS = 16
TOKENS = 16384
BLOCK = 1024


def _router_kernel(x_ref, wt_ref, w_out_ref, i_out_ref):
    logits = jnp.dot(x_ref[...], wt_ref[...],
                     preferred_element_type=jnp.float32)
    lane = jax.lax.broadcasted_iota(jnp.int32, logits.shape, 1)
    m1 = jnp.max(logits, axis=1, keepdims=True)
    i1 = jnp.min(jnp.where(logits == m1, lane, NUM_EXPERTS), axis=1,
                 keepdims=True)
    masked = jnp.where(lane == i1, -jnp.inf, logits)
    m2 = jnp.max(masked, axis=1, keepdims=True)
    i2 = jnp.min(jnp.where(masked == m2, lane, NUM_EXPERTS), axis=1,
                 keepdims=True)
    r = jnp.exp(m2 - m1)
    w1 = 1.0 / (1.0 + r)
    w_out_ref[...] = jnp.concatenate([w1, 1.0 - w1], axis=1)
    i_out_ref[...] = jnp.concatenate([i1, i2], axis=1)


@jax.jit
def kernel(x, W):
    grid = (TOKENS // BLOCK,)
    w_out, i_out = pl.pallas_call(
        _router_kernel,
        grid=grid,
        in_specs=[
            pl.BlockSpec((BLOCK, HIDDEN), lambda i: (i, 0)),
            pl.BlockSpec((HIDDEN, NUM_EXPERTS), lambda i: (0, 0)),
        ],
        out_specs=[
            pl.BlockSpec((BLOCK, 2), lambda i: (i, 0)),
            pl.BlockSpec((BLOCK, 2), lambda i: (i, 0)),
        ],
        out_shape=[
            jax.ShapeDtypeStruct((TOKENS, 2), jnp.float32),
            jax.ShapeDtypeStruct((TOKENS, 2), jnp.int32),
        ],
        compiler_params=pltpu.CompilerParams(
            dimension_semantics=("arbitrary",),
        ),
    )(x, W.T)
    return (w_out, i_out)
